# baseline (speedup 1.0000x reference)
import functools

import jax
import jax.numpy as jnp
from jax import lax
from jax.experimental import pallas as pl
from jax.experimental.pallas import tpu as pltpu
from jax.experimental.pallas import tpu_sc as plsc

N = 10000
E = 320000
D = 128
B = 16
H = 128
G = 16

NC = 2
NS = 16
L = 16
NW = NC * NS
CH = 128
CPT = 80
CPT0 = 160
CPT1 = 0
EPT = CH * CPT
EP = NW * EPT
NP = 10112
RPT = NP // NS

_mesh = plsc.VectorSubcoreMesh(core_axis_name="c", subcore_axis_name="s")
_sc_params = pltpu.CompilerParams(needs_layout_passes=False,
                                  use_tc_tiling_on_sc=False)


@functools.partial(
    pl.kernel,
    mesh=_mesh,
    out_type=[
        jax.ShapeDtypeStruct((EP, D), jnp.float32),
        jax.ShapeDtypeStruct((EP, D), jnp.float32),
        jax.ShapeDtypeStruct((EP, G), jnp.float32),
    ],
    scratch_types=[
        [pltpu.VMEM((CH,), jnp.int32)] * 2,
        [pltpu.VMEM((CH,), jnp.int32)] * 2,
        [pltpu.VMEM((CH, D), jnp.float32)] * 2,
        [pltpu.VMEM((CH, D), jnp.float32)] * 2,
        pltpu.VMEM((3 * N,), jnp.float32),
        [pltpu.VMEM((CH, G), jnp.float32)] * 2,
        [pltpu.SemaphoreType.DMA] * 2,
        [pltpu.SemaphoreType.DMA] * 2,
        [pltpu.SemaphoreType.DMA] * 2,
        [pltpu.SemaphoreType.DMA] * 2,
        [pltpu.SemaphoreType.DMA] * 2,
        [pltpu.SemaphoreType.DMA] * 2,
        [pltpu.SemaphoreType.DMA] * 2,
    ],
    compiler_params=_sc_params,
)
def _gather_kernel(x_hbm, posf_hbm, row_hbm, col_hbm,
                   xr_hbm, xc_hbm, geo_hbm,
                   rowv, colv, xrv, xcv, posv, geov,
                   semir, semic, semgr, semgc, semwr, semwc, semwg):
    cid = lax.axis_index("c")
    sid = lax.axis_index("s")
    cpt_l = jnp.where(cid == 0, CPT0, CPT1)
    base_chunk = jnp.where(cid == 0, sid * CPT0, NS * CPT0 + sid * CPT1)
    tbase = base_chunk * CH
    zero16 = jnp.zeros((L,), jnp.float32)
    for b in range(2):
        for t in range(CH):
            geov[b][t, :] = zero16
    lane = lax.iota(jnp.int32, L)

    def idx_start(cb, b):
        ebase = tbase + cb * CH
        pltpu.async_copy(row_hbm.at[pl.ds(ebase, CH)], rowv[b], semir[b])
        pltpu.async_copy(col_hbm.at[pl.ds(ebase, CH)], colv[b], semic[b])

    def idx_wait(cb, b):
        ebase = tbase + cb * CH
        pltpu.make_async_copy(row_hbm.at[pl.ds(ebase, CH)], rowv[b], semir[b]).wait()
        pltpu.make_async_copy(col_hbm.at[pl.ds(ebase, CH)], colv[b], semic[b]).wait()

    def wb_wait(cb, b):
        ebase = tbase + cb * CH
        pltpu.make_async_copy(xrv[b], xr_hbm.at[pl.ds(ebase, CH)], semwr[b]).wait()
        pltpu.make_async_copy(xcv[b], xc_hbm.at[pl.ds(ebase, CH)], semwc[b]).wait()
        pltpu.make_async_copy(geov[b], geo_hbm.at[pl.ds(ebase, CH)], semwg[b]).wait()

    @pl.when(cpt_l > 0)
    def _():
        pltpu.sync_copy(posf_hbm, posv)
        idx_start(0, 0)
        idx_start(1, 1)

    def chunk(cb, carry):
        b = lax.rem(cb, 2)
        ebase = tbase + cb * CH

        def on_buf(bb):
            idx_wait(cb, bb)

            @pl.when(cb >= 2)
            def _():
                wb_wait(cb, bb)

            gr = pltpu.async_copy(x_hbm.at[rowv[bb]], xrv[bb], semgr[bb])
            gc = pltpu.async_copy(x_hbm.at[colv[bb]], xcv[bb], semgc[bb])

            for j in range(CH // L):
                r3 = rowv[bb][pl.ds(j * L, L)] * 3
                c3 = colv[bb][pl.ds(j * L, L)] * 3
                prx = plsc.load_gather(posv, [r3])
                pry = plsc.load_gather(posv, [r3 + 1])
                prz = plsc.load_gather(posv, [r3 + 2])
                pcx = plsc.load_gather(posv, [c3])
                pcy = plsc.load_gather(posv, [c3 + 1])
                pcz = plsc.load_gather(posv, [c3 + 2])
                dx = prx - pcx
                dy = pry - pcy
                dz = prz - pcz
                dsq = dx * dx + dy * dy + dz * dz
                rows = lane + j * L
                plsc.store_scatter(geov[bb], [rows, jnp.full((L,), 0, jnp.int32)], dsq)
                plsc.store_scatter(geov[bb], [rows, jnp.full((L,), 1, jnp.int32)], dx)
                plsc.store_scatter(geov[bb], [rows, jnp.full((L,), 2, jnp.int32)], dy)
                plsc.store_scatter(geov[bb], [rows, jnp.full((L,), 3, jnp.int32)], dz)
            pltpu.async_copy(geov[bb], geo_hbm.at[pl.ds(ebase, CH)], semwg[bb])
            gr.wait()
            gc.wait()

            @pl.when(cb + 2 < cpt_l)
            def _():
                idx_start(cb + 2, bb)

            pltpu.async_copy(xrv[bb], xr_hbm.at[pl.ds(ebase, CH)], semwr[bb])
            pltpu.async_copy(xcv[bb], xc_hbm.at[pl.ds(ebase, CH)], semwc[bb])

        @pl.when(b == 0)
        def _():
            on_buf(0)

        @pl.when(b == 1)
        def _():
            on_buf(1)

        return carry

    lax.fori_loop(0, cpt_l, chunk, 0)

    @pl.when(cpt_l > 0)
    def _():
        wb_wait(cpt_l - 2, 0)
        wb_wait(cpt_l - 1, 1)


@functools.partial(
    pl.kernel,
    mesh=_mesh,
    out_type=[
        jax.ShapeDtypeStruct((NC, NP, D), jnp.float32),
        jax.ShapeDtypeStruct((NC, NP, G), jnp.float32),
    ],
    scratch_types=[
        [pltpu.VMEM((CH,), jnp.int32)] * 2,
        [pltpu.VMEM((CH, D), jnp.float32)] * 2,
        [pltpu.VMEM((CH, G), jnp.float32)] * 2,
        pltpu.VMEM_SHARED((NP, D), jnp.float32),
        pltpu.VMEM_SHARED((NP, G), jnp.float32),
        [pltpu.SemaphoreType.DMA] * 2,
        [pltpu.SemaphoreType.DMA] * 2,
        [pltpu.SemaphoreType.DMA] * 2,
    ],
    compiler_params=_sc_params,
)
def _scatter_kernel(col_hbm, msg_hbm, pu_hbm, zx_hbm, zp_hbm,
                    px_hbm, pp_hbm,
                    colv, msgv, puv, accx, accp, semc, semm, semp):
    cid = lax.axis_index("c")
    sid = lax.axis_index("s")
    wid = sid * NC + cid
    rbase = sid * RPT
    pltpu.sync_copy(zx_hbm.at[pl.ds(rbase, RPT)], accx.at[pl.ds(rbase, RPT)])
    pltpu.sync_copy(zp_hbm.at[pl.ds(rbase, RPT)], accp.at[pl.ds(rbase, RPT)])
    plsc.subcore_barrier()

    def cstart(cb, b):
        ebase = wid * EPT + cb * CH
        pltpu.async_copy(col_hbm.at[pl.ds(ebase, CH)], colv[b], semc[b])
        pltpu.async_copy(msg_hbm.at[pl.ds(ebase, CH)], msgv[b], semm[b])
        pltpu.async_copy(pu_hbm.at[pl.ds(ebase, CH)], puv[b], semp[b])

    def cwait(cb, b):
        ebase = wid * EPT + cb * CH
        pltpu.make_async_copy(col_hbm.at[pl.ds(ebase, CH)], colv[b], semc[b]).wait()
        pltpu.make_async_copy(msg_hbm.at[pl.ds(ebase, CH)], msgv[b], semm[b]).wait()
        pltpu.make_async_copy(pu_hbm.at[pl.ds(ebase, CH)], puv[b], semp[b]).wait()

    cstart(0, 0)
    cstart(1, 1)

    def chunk(cb, carry):
        b = lax.rem(cb, 2)

        def on_buf(bb):
            cwait(cb, bb)
            pltpu.sync_copy(msgv[bb], accx.at[colv[bb]], add=True)
            pltpu.sync_copy(puv[bb], accp.at[colv[bb]], add=True)

            @pl.when(cb + 2 < CPT)
            def _():
                cstart(cb + 2, bb)

        @pl.when(b == 0)
        def _():
            on_buf(0)

        @pl.when(b == 1)
        def _():
            on_buf(1)

        return carry

    lax.fori_loop(0, CPT, chunk, 0)
    plsc.subcore_barrier()
    pltpu.sync_copy(accx.at[pl.ds(rbase, RPT)], px_hbm.at[cid].at[pl.ds(rbase, RPT)])
    pltpu.sync_copy(accp.at[pl.ds(rbase, RPT)], pp_hbm.at[cid].at[pl.ds(rbase, RPT)])


BE = 1280


def _mlp_body(xr, xc, ea, geo, w1a, w1b, w1c, g1, b1, wx2, bx2, wp2, bp2,
              we2, be2, msg_o, pu_o, eu_o):
    f32 = jnp.float32
    h = (jnp.dot(xr[...], w1a[...], preferred_element_type=f32)
         + jnp.dot(xc[...], w1b[...], preferred_element_type=f32)
         + jnp.dot(ea[...], w1c[...], preferred_element_type=f32)
         + jnp.dot(geo[...], g1[...], preferred_element_type=f32)
         + b1[...])
    h = h * jax.nn.sigmoid(h)
    msg_o[...] = jnp.dot(h[:, :H], wx2[...], preferred_element_type=f32) + bx2[...]
    wp = jnp.dot(h[:, H:2 * H], wp2[...], preferred_element_type=f32) + bp2[...]
    eu_o[...] = jnp.dot(h[:, 2 * H:], we2[...], preferred_element_type=f32) + be2[...]
    colid = lax.broadcasted_iota(jnp.int32, (1, G), 1)
    relmask = jnp.where((colid >= 1) & (colid <= 3), 1.0, 0.0).astype(f32)
    pu_o[...] = wp * (geo[...] * relmask)


def _full(shape):
    return pl.BlockSpec(shape, lambda i: (0,) * len(shape))


_mlp_call = pl.pallas_call(
    _mlp_body,
    grid=(E // BE,),
    in_specs=[
        pl.BlockSpec((BE, D), lambda i: (i, 0)),
        pl.BlockSpec((BE, D), lambda i: (i, 0)),
        pl.BlockSpec((BE, B), lambda i: (i, 0)),
        pl.BlockSpec((BE, G), lambda i: (i, 0)),
        _full((D, 3 * H)),
        _full((D, 3 * H)),
        _full((B, 3 * H)),
        _full((G, 3 * H)),
        _full((1, 3 * H)),
        _full((H, D)),
        _full((1, D)),
        _full((H, 1)),
        _full((1, 1)),
        _full((H, B)),
        _full((1, B)),
    ],
    out_specs=[
        pl.BlockSpec((BE, D), lambda i: (i, 0)),
        pl.BlockSpec((BE, G), lambda i: (i, 0)),
        pl.BlockSpec((BE, B), lambda i: (i, 0)),
    ],
    out_shape=[
        jax.ShapeDtypeStruct((EP, D), jnp.float32),
        jax.ShapeDtypeStruct((EP, G), jnp.float32),
        jax.ShapeDtypeStruct((E, B), jnp.float32),
    ],
)

BN = 2000


def _combine_body(px, pp, ax_o, ap_o):
    ax_o[...] = px[0] + px[1]
    ap_o[...] = pp[0] + pp[1]


_combine_call = pl.pallas_call(
    _combine_body,
    grid=(N // BN,),
    in_specs=[
        pl.BlockSpec((NC, BN, D), lambda i: (0, i, 0)),
        pl.BlockSpec((NC, BN, G), lambda i: (0, i, 0)),
    ],
    out_specs=[
        pl.BlockSpec((BN, D), lambda i: (i, 0)),
        pl.BlockSpec((BN, G), lambda i: (i, 0)),
    ],
    out_shape=[
        jax.ShapeDtypeStruct((N, D), jnp.float32),
        jax.ShapeDtypeStruct((N, G), jnp.float32),
    ],
)


def kernel(x, pos, edge_index, edge_attr, Wx1, bx1, Wx2, bx2,
           Wp1, bp1, Wp2, bp2, We1, be1, We2, be2):
    pad = EP - E
    rowp = jnp.concatenate([edge_index[0], jnp.zeros((pad,), jnp.int32)])
    colg = jnp.concatenate([edge_index[1], jnp.zeros((pad,), jnp.int32)])
    colp = jnp.concatenate([edge_index[1], jnp.full((pad,), N, jnp.int32)])
    posf = pos.reshape(-1)

    xr, xc, geo = _gather_kernel(x, posf, rowp, colg)

    w1cat = jnp.concatenate([Wx1, Wp1, We1], axis=1)
    b1cat = jnp.concatenate([bx1, bp1, be1]).reshape(1, 3 * H)
    w1a = w1cat[:D]
    w1b = w1cat[D:2 * D]
    w1c = w1cat[2 * D:2 * D + B]
    g1 = jnp.zeros((G, 3 * H), jnp.float32).at[0].set(w1cat[2 * D + B])

    msg, pu, eu = _mlp_call(
        xr, xc, edge_attr, geo, w1a, w1b, w1c, g1, b1cat,
        Wx2, bx2.reshape(1, D), Wp2, bp2.reshape(1, 1),
        We2, be2.reshape(1, B))

    zx = jnp.zeros((NP, D), jnp.float32)
    zp = jnp.zeros((NP, G), jnp.float32)
    px, pp = _scatter_kernel(colp, msg, pu, zx, zp)
    aggx, aggp = _combine_call(px, pp)
    return aggx, aggp[:, 1:4], eu


# speedup vs baseline: 1.6471x; 1.6471x over previous
import functools

import jax
import jax.numpy as jnp
from jax import lax
from jax.experimental import pallas as pl
from jax.experimental.pallas import tpu as pltpu
from jax.experimental.pallas import tpu_sc as plsc

N = 10000
E = 320000
D = 128
B = 16
H = 128
G = 16

NC = 2
NS = 16
L = 16
NW = NC * NS
CH = 64
CPT = 160
SCH = 128
SCPT = 80
EPT = CH * CPT
EP = NW * EPT
NP = 10112
RPT = NP // NS
NXT = N // NS

_mesh = plsc.VectorSubcoreMesh(core_axis_name="c", subcore_axis_name="s")
_sc_params = pltpu.CompilerParams(needs_layout_passes=False,
                                  use_tc_tiling_on_sc=False)


@functools.partial(
    pl.kernel,
    mesh=_mesh,
    out_type=[
        jax.ShapeDtypeStruct((EP, D), jnp.float32),
        jax.ShapeDtypeStruct((EP, D), jnp.float32),
        jax.ShapeDtypeStruct((EP, G), jnp.float32),
        jax.ShapeDtypeStruct((EP, G), jnp.float32),
    ],
    scratch_types=[
        [pltpu.VMEM((CH,), jnp.int32)] * 2,
        [pltpu.VMEM((CH,), jnp.int32)] * 2,
        [pltpu.VMEM((CH, D), jnp.float32)] * 2,
        [pltpu.VMEM((CH, D), jnp.float32)] * 2,
        [pltpu.VMEM((CH, G), jnp.float32)] * 2,
        [pltpu.VMEM((CH, G), jnp.float32)] * 2,
        pltpu.VMEM_SHARED((N, D), jnp.float32),
        pltpu.VMEM_SHARED((N, G), jnp.float32),
        [pltpu.SemaphoreType.DMA] * 2,
        [pltpu.SemaphoreType.DMA] * 2,
        [pltpu.SemaphoreType.DMA] * 2,
        [pltpu.SemaphoreType.DMA] * 2,
        [pltpu.SemaphoreType.DMA] * 2,
        [pltpu.SemaphoreType.DMA] * 2,
        [pltpu.SemaphoreType.DMA] * 2,
        [pltpu.SemaphoreType.DMA] * 2,
        [pltpu.SemaphoreType.DMA] * 2,
        [pltpu.SemaphoreType.DMA] * 2,
    ],
    compiler_params=_sc_params,
)
def _gather_kernel(x_hbm, posp_hbm, row_hbm, col_hbm,
                   xr_hbm, xc_hbm, pr_hbm, pc_hbm,
                   rowv, colv, xrv, xcv, prv, pcv, xspm, pspm,
                   semir, semic, semxr, semxc, sempr, sempc,
                   semwxr, semwxc, semwpr, semwpc):
    cid = lax.axis_index("c")
    sid = lax.axis_index("s")
    wid = sid * NC + cid
    tbase = wid * EPT
    pltpu.sync_copy(x_hbm.at[pl.ds(sid * NXT, NXT)], xspm.at[pl.ds(sid * NXT, NXT)])
    pltpu.sync_copy(posp_hbm.at[pl.ds(sid * NXT, NXT)], pspm.at[pl.ds(sid * NXT, NXT)])
    plsc.subcore_barrier()

    def idx_start(cb, b):
        ebase = tbase + cb * CH
        pltpu.async_copy(row_hbm.at[pl.ds(ebase, CH)], rowv[b], semir[b])
        pltpu.async_copy(col_hbm.at[pl.ds(ebase, CH)], colv[b], semic[b])

    def idx_wait(cb, b):
        ebase = tbase + cb * CH
        pltpu.make_async_copy(row_hbm.at[pl.ds(ebase, CH)], rowv[b], semir[b]).wait()
        pltpu.make_async_copy(col_hbm.at[pl.ds(ebase, CH)], colv[b], semic[b]).wait()

    def wb_start(cb, b):
        ebase = tbase + cb * CH
        pltpu.async_copy(xrv[b], xr_hbm.at[pl.ds(ebase, CH)], semwxr[b])
        pltpu.async_copy(xcv[b], xc_hbm.at[pl.ds(ebase, CH)], semwxc[b])
        pltpu.async_copy(prv[b], pr_hbm.at[pl.ds(ebase, CH)], semwpr[b])
        pltpu.async_copy(pcv[b], pc_hbm.at[pl.ds(ebase, CH)], semwpc[b])

    def wb_wait(cb, b):
        ebase = tbase + cb * CH
        pltpu.make_async_copy(xrv[b], xr_hbm.at[pl.ds(ebase, CH)], semwxr[b]).wait()
        pltpu.make_async_copy(xcv[b], xc_hbm.at[pl.ds(ebase, CH)], semwxc[b]).wait()
        pltpu.make_async_copy(prv[b], pr_hbm.at[pl.ds(ebase, CH)], semwpr[b]).wait()
        pltpu.make_async_copy(pcv[b], pc_hbm.at[pl.ds(ebase, CH)], semwpc[b]).wait()

    idx_start(0, 0)
    idx_start(1, 1)

    def chunk(cb, carry):
        b = lax.rem(cb, 2)

        def on_buf(bb):
            idx_wait(cb, bb)

            @pl.when(cb >= 2)
            def _():
                wb_wait(cb, bb)

            g1 = pltpu.async_copy(xspm.at[rowv[bb]], xrv[bb], semxr[bb])
            g2 = pltpu.async_copy(xspm.at[colv[bb]], xcv[bb], semxc[bb])
            g3 = pltpu.async_copy(pspm.at[rowv[bb]], prv[bb], sempr[bb])
            g4 = pltpu.async_copy(pspm.at[colv[bb]], pcv[bb], sempc[bb])
            g1.wait()
            g2.wait()
            g3.wait()
            g4.wait()

            @pl.when(cb + 2 < CPT)
            def _():
                idx_start(cb + 2, bb)

            wb_start(cb, bb)

        @pl.when(b == 0)
        def _():
            on_buf(0)

        @pl.when(b == 1)
        def _():
            on_buf(1)

        return carry

    lax.fori_loop(0, CPT, chunk, 0)
    wb_wait(CPT - 2, 0)
    wb_wait(CPT - 1, 1)


@functools.partial(
    pl.kernel,
    mesh=_mesh,
    out_type=[
        jax.ShapeDtypeStruct((NC, NP, D), jnp.float32),
        jax.ShapeDtypeStruct((NC, NP, G), jnp.float32),
    ],
    scratch_types=[
        [pltpu.VMEM((SCH,), jnp.int32)] * 2,
        [pltpu.VMEM((SCH, D), jnp.float32)] * 2,
        [pltpu.VMEM((SCH, G), jnp.float32)] * 2,
        pltpu.VMEM_SHARED((NP, D), jnp.float32),
        pltpu.VMEM_SHARED((NP, G), jnp.float32),
        [pltpu.SemaphoreType.DMA] * 2,
        [pltpu.SemaphoreType.DMA] * 2,
        [pltpu.SemaphoreType.DMA] * 2,
    ],
    compiler_params=_sc_params,
)
def _scatter_kernel(col_hbm, msg_hbm, pu_hbm, zx_hbm, zp_hbm,
                    px_hbm, pp_hbm,
                    colv, msgv, puv, accx, accp, semc, semm, semp):
    cid = lax.axis_index("c")
    sid = lax.axis_index("s")
    wid = sid * NC + cid
    rbase = sid * RPT
    pltpu.sync_copy(zx_hbm.at[pl.ds(rbase, RPT)], accx.at[pl.ds(rbase, RPT)])
    pltpu.sync_copy(zp_hbm.at[pl.ds(rbase, RPT)], accp.at[pl.ds(rbase, RPT)])
    plsc.subcore_barrier()

    def cstart(cb, b):
        ebase = wid * EPT + cb * SCH
        pltpu.async_copy(col_hbm.at[pl.ds(ebase, SCH)], colv[b], semc[b])
        pltpu.async_copy(msg_hbm.at[pl.ds(ebase, SCH)], msgv[b], semm[b])
        pltpu.async_copy(pu_hbm.at[pl.ds(ebase, SCH)], puv[b], semp[b])

    def cwait(cb, b):
        ebase = wid * EPT + cb * SCH
        pltpu.make_async_copy(col_hbm.at[pl.ds(ebase, SCH)], colv[b], semc[b]).wait()
        pltpu.make_async_copy(msg_hbm.at[pl.ds(ebase, SCH)], msgv[b], semm[b]).wait()
        pltpu.make_async_copy(pu_hbm.at[pl.ds(ebase, SCH)], puv[b], semp[b]).wait()

    cstart(0, 0)
    cstart(1, 1)

    def chunk(cb, carry):
        b = lax.rem(cb, 2)

        def on_buf(bb):
            cwait(cb, bb)
            pltpu.sync_copy(msgv[bb], accx.at[colv[bb]], add=True)
            pltpu.sync_copy(puv[bb], accp.at[colv[bb]], add=True)

            @pl.when(cb + 2 < SCPT)
            def _():
                cstart(cb + 2, bb)

        @pl.when(b == 0)
        def _():
            on_buf(0)

        @pl.when(b == 1)
        def _():
            on_buf(1)

        return carry

    lax.fori_loop(0, SCPT, chunk, 0)
    plsc.subcore_barrier()
    pltpu.sync_copy(accx.at[pl.ds(rbase, RPT)], px_hbm.at[cid].at[pl.ds(rbase, RPT)])
    pltpu.sync_copy(accp.at[pl.ds(rbase, RPT)], pp_hbm.at[cid].at[pl.ds(rbase, RPT)])


BE = 1280


def _mlp_body(xr, xc, ea, pr, pc, w1a, w1b, w1c, s1, b1, wx2, bx2, wp2, bp2,
              we2, be2, msg_o, pu_o, eu_o):
    f32 = jnp.float32
    dr = pr[...] - pc[...]
    h = (jnp.dot(xr[...], w1a[...], preferred_element_type=f32)
         + jnp.dot(xc[...], w1b[...], preferred_element_type=f32)
         + jnp.dot(ea[...], w1c[...], preferred_element_type=f32)
         + jnp.dot(dr * dr, s1[...], preferred_element_type=f32)
         + b1[...])
    h = h * jax.nn.sigmoid(h)
    msg_o[...] = jnp.dot(h[:, :H], wx2[...], preferred_element_type=f32) + bx2[...]
    wp = jnp.dot(h[:, H:2 * H], wp2[...], preferred_element_type=f32) + bp2[...]
    eu_o[...] = jnp.dot(h[:, 2 * H:], we2[...], preferred_element_type=f32) + be2[...]
    colid = lax.broadcasted_iota(jnp.int32, (1, G), 1)
    relmask = jnp.where(colid < 3, 1.0, 0.0).astype(f32)
    pu_o[...] = wp * (dr * relmask)


def _full(shape):
    return pl.BlockSpec(shape, lambda i: (0,) * len(shape))


_mlp_call = pl.pallas_call(
    _mlp_body,
    grid=(E // BE,),
    in_specs=[
        pl.BlockSpec((BE, D), lambda i: (i, 0)),
        pl.BlockSpec((BE, D), lambda i: (i, 0)),
        pl.BlockSpec((BE, B), lambda i: (i, 0)),
        pl.BlockSpec((BE, G), lambda i: (i, 0)),
        pl.BlockSpec((BE, G), lambda i: (i, 0)),
        _full((D, 3 * H)),
        _full((D, 3 * H)),
        _full((B, 3 * H)),
        _full((G, 3 * H)),
        _full((1, 3 * H)),
        _full((H, D)),
        _full((1, D)),
        _full((H, 1)),
        _full((1, 1)),
        _full((H, B)),
        _full((1, B)),
    ],
    out_specs=[
        pl.BlockSpec((BE, D), lambda i: (i, 0)),
        pl.BlockSpec((BE, G), lambda i: (i, 0)),
        pl.BlockSpec((BE, B), lambda i: (i, 0)),
    ],
    out_shape=[
        jax.ShapeDtypeStruct((EP, D), jnp.float32),
        jax.ShapeDtypeStruct((EP, G), jnp.float32),
        jax.ShapeDtypeStruct((E, B), jnp.float32),
    ],
)

BN = 2000


def _combine_body(px, pp, ax_o, ap_o):
    ax_o[...] = px[0] + px[1]
    ap_o[...] = pp[0] + pp[1]


_combine_call = pl.pallas_call(
    _combine_body,
    grid=(N // BN,),
    in_specs=[
        pl.BlockSpec((NC, BN, D), lambda i: (0, i, 0)),
        pl.BlockSpec((NC, BN, G), lambda i: (0, i, 0)),
    ],
    out_specs=[
        pl.BlockSpec((BN, D), lambda i: (i, 0)),
        pl.BlockSpec((BN, G), lambda i: (i, 0)),
    ],
    out_shape=[
        jax.ShapeDtypeStruct((N, D), jnp.float32),
        jax.ShapeDtypeStruct((N, G), jnp.float32),
    ],
)


def kernel(x, pos, edge_index, edge_attr, Wx1, bx1, Wx2, bx2,
           Wp1, bp1, Wp2, bp2, We1, be1, We2, be2):
    pad = EP - E
    rowp = jnp.concatenate([edge_index[0], jnp.zeros((pad,), jnp.int32)])
    colg = jnp.concatenate([edge_index[1], jnp.zeros((pad,), jnp.int32)])
    colp = jnp.concatenate([edge_index[1], jnp.full((pad,), N, jnp.int32)])
    posp = jnp.zeros((N, G), jnp.float32).at[:, :3].set(pos)

    xr, xc, pr, pc = _gather_kernel(x, posp, rowp, colg)

    w1cat = jnp.concatenate([Wx1, Wp1, We1], axis=1)
    b1cat = jnp.concatenate([bx1, bp1, be1]).reshape(1, 3 * H)
    w1a = w1cat[:D]
    w1b = w1cat[D:2 * D]
    w1c = w1cat[2 * D:2 * D + B]
    w1d = w1cat[2 * D + B]
    s1 = jnp.zeros((G, 3 * H), jnp.float32).at[0].set(w1d).at[1].set(w1d).at[2].set(w1d)

    msg, pu, eu = _mlp_call(
        xr, xc, edge_attr, pr, pc, w1a, w1b, w1c, s1, b1cat,
        Wx2, bx2.reshape(1, D), Wp2, bp2.reshape(1, 1),
        We2, be2.reshape(1, B))

    zx = jnp.zeros((NP, D), jnp.float32)
    zp = jnp.zeros((NP, G), jnp.float32)
    px, pp = _scatter_kernel(colp, msg, pu, zx, zp)
    aggx, aggp = _combine_call(px, pp)
    return aggx, aggp[:, :3], eu


# speedup vs baseline: 1.8172x; 1.1033x over previous
import functools

import jax
import jax.numpy as jnp
from jax import lax
from jax.experimental import pallas as pl
from jax.experimental.pallas import tpu as pltpu
from jax.experimental.pallas import tpu_sc as plsc

N = 10000
E = 320000
D = 128
B = 16
H = 128
G = 16

NC = 2
NS = 16
L = 16
NW = NC * NS
CH = 64
CPT = 160
SCH = 128
SCPT = 80
EPT = CH * CPT
EP = NW * EPT
NP = 10112
RPT = NP // NS
NXT = N // NS

_mesh = plsc.VectorSubcoreMesh(core_axis_name="c", subcore_axis_name="s")
_sc_params = pltpu.CompilerParams(needs_layout_passes=False,
                                  use_tc_tiling_on_sc=False)


@functools.partial(
    pl.kernel,
    mesh=_mesh,
    out_type=[
        jax.ShapeDtypeStruct((EP, D), jnp.float32),
        jax.ShapeDtypeStruct((EP, D), jnp.float32),
        jax.ShapeDtypeStruct((EP, G), jnp.float32),
    ],
    scratch_types=[
        [pltpu.VMEM((CH,), jnp.int32)] * 2,
        [pltpu.VMEM((CH,), jnp.int32)] * 2,
        [pltpu.VMEM((CH, D), jnp.float32)] * 2,
        [pltpu.VMEM((CH, D), jnp.float32)] * 2,
        [pltpu.VMEM((CH, G), jnp.float32)] * 2,
        [pltpu.VMEM((CH, G), jnp.float32)] * 2,
        pltpu.VMEM_SHARED((N, D), jnp.float32),
        pltpu.VMEM_SHARED((N, G), jnp.float32),
        [pltpu.SemaphoreType.DMA] * 2,
        [pltpu.SemaphoreType.DMA] * 2,
        [pltpu.SemaphoreType.DMA] * 2,
        [pltpu.SemaphoreType.DMA] * 2,
        [pltpu.SemaphoreType.DMA] * 2,
        [pltpu.SemaphoreType.DMA] * 2,
        [pltpu.SemaphoreType.DMA] * 2,
        [pltpu.SemaphoreType.DMA] * 2,
        [pltpu.SemaphoreType.DMA] * 2,
    ],
    compiler_params=_sc_params,
)
def _gather_kernel(x_hbm, posp_hbm, row_hbm, col_hbm,
                   xr_hbm, xc_hbm, dr_hbm,
                   rowv, colv, xrv, xcv, prv, pcv, xspm, pspm,
                   semir, semic, semxr, semxc, sempr, sempc,
                   semwxr, semwxc, semwpr):
    cid = lax.axis_index("c")
    sid = lax.axis_index("s")
    wid = sid * NC + cid
    tbase = wid * EPT
    pltpu.sync_copy(x_hbm.at[pl.ds(sid * NXT, NXT)], xspm.at[pl.ds(sid * NXT, NXT)])
    pltpu.sync_copy(posp_hbm.at[pl.ds(sid * NXT, NXT)], pspm.at[pl.ds(sid * NXT, NXT)])
    plsc.subcore_barrier()

    def idx_start(cb, b):
        ebase = tbase + cb * CH
        pltpu.async_copy(row_hbm.at[pl.ds(ebase, CH)], rowv[b], semir[b])
        pltpu.async_copy(col_hbm.at[pl.ds(ebase, CH)], colv[b], semic[b])

    def idx_wait(cb, b):
        ebase = tbase + cb * CH
        pltpu.make_async_copy(row_hbm.at[pl.ds(ebase, CH)], rowv[b], semir[b]).wait()
        pltpu.make_async_copy(col_hbm.at[pl.ds(ebase, CH)], colv[b], semic[b]).wait()

    def wb_start(cb, b):
        ebase = tbase + cb * CH
        pltpu.async_copy(xrv[b], xr_hbm.at[pl.ds(ebase, CH)], semwxr[b])
        pltpu.async_copy(xcv[b], xc_hbm.at[pl.ds(ebase, CH)], semwxc[b])
        pltpu.async_copy(prv[b], dr_hbm.at[pl.ds(ebase, CH)], semwpr[b])

    def wb_wait(cb, b):
        ebase = tbase + cb * CH
        pltpu.make_async_copy(xrv[b], xr_hbm.at[pl.ds(ebase, CH)], semwxr[b]).wait()
        pltpu.make_async_copy(xcv[b], xc_hbm.at[pl.ds(ebase, CH)], semwxc[b]).wait()
        pltpu.make_async_copy(prv[b], dr_hbm.at[pl.ds(ebase, CH)], semwpr[b]).wait()

    idx_start(0, 0)
    idx_start(1, 1)

    def chunk(cb, carry):
        b = lax.rem(cb, 2)

        def on_buf(bb):
            idx_wait(cb, bb)

            @pl.when(cb >= 2)
            def _():
                wb_wait(cb, bb)

            g1 = pltpu.async_copy(xspm.at[rowv[bb]], xrv[bb], semxr[bb])
            g2 = pltpu.async_copy(xspm.at[colv[bb]], xcv[bb], semxc[bb])
            g3 = pltpu.async_copy(pspm.at[rowv[bb]], prv[bb], sempr[bb])
            g4 = pltpu.async_copy(pspm.at[colv[bb]], pcv[bb], sempc[bb])
            g3.wait()
            g4.wait()
            for e in range(CH):
                prv[bb][e, :] = prv[bb][e, :] - pcv[bb][e, :]
            g1.wait()
            g2.wait()

            @pl.when(cb + 2 < CPT)
            def _():
                idx_start(cb + 2, bb)

            wb_start(cb, bb)

        @pl.when(b == 0)
        def _():
            on_buf(0)

        @pl.when(b == 1)
        def _():
            on_buf(1)

        return carry

    lax.fori_loop(0, CPT, chunk, 0)
    wb_wait(CPT - 2, 0)
    wb_wait(CPT - 1, 1)


@functools.partial(
    pl.kernel,
    mesh=_mesh,
    out_type=[
        jax.ShapeDtypeStruct((NC, NP, D), jnp.float32),
        jax.ShapeDtypeStruct((NC, NP, G), jnp.float32),
    ],
    scratch_types=[
        [pltpu.VMEM((SCH,), jnp.int32)] * 2,
        [pltpu.VMEM((SCH, D), jnp.float32)] * 2,
        [pltpu.VMEM((SCH, G), jnp.float32)] * 2,
        pltpu.VMEM_SHARED((NP, D), jnp.float32),
        pltpu.VMEM_SHARED((NP, G), jnp.float32),
        [pltpu.SemaphoreType.DMA] * 2,
        [pltpu.SemaphoreType.DMA] * 2,
        [pltpu.SemaphoreType.DMA] * 2,
    ],
    compiler_params=_sc_params,
)
def _scatter_kernel(col_hbm, msg_hbm, pu_hbm, zx_hbm, zp_hbm,
                    px_hbm, pp_hbm,
                    colv, msgv, puv, accx, accp, semc, semm, semp):
    cid = lax.axis_index("c")
    sid = lax.axis_index("s")
    wid = sid * NC + cid
    rbase = sid * RPT
    pltpu.sync_copy(zx_hbm.at[pl.ds(rbase, RPT)], accx.at[pl.ds(rbase, RPT)])
    pltpu.sync_copy(zp_hbm.at[pl.ds(rbase, RPT)], accp.at[pl.ds(rbase, RPT)])
    plsc.subcore_barrier()

    def cstart(cb, b):
        ebase = wid * EPT + cb * SCH
        pltpu.async_copy(col_hbm.at[pl.ds(ebase, SCH)], colv[b], semc[b])
        pltpu.async_copy(msg_hbm.at[pl.ds(ebase, SCH)], msgv[b], semm[b])
        pltpu.async_copy(pu_hbm.at[pl.ds(ebase, SCH)], puv[b], semp[b])

    def cwait(cb, b):
        ebase = wid * EPT + cb * SCH
        pltpu.make_async_copy(col_hbm.at[pl.ds(ebase, SCH)], colv[b], semc[b]).wait()
        pltpu.make_async_copy(msg_hbm.at[pl.ds(ebase, SCH)], msgv[b], semm[b]).wait()
        pltpu.make_async_copy(pu_hbm.at[pl.ds(ebase, SCH)], puv[b], semp[b]).wait()

    cstart(0, 0)
    cstart(1, 1)

    def chunk(cb, carry):
        b = lax.rem(cb, 2)

        def on_buf(bb):
            cwait(cb, bb)
            pltpu.sync_copy(msgv[bb], accx.at[colv[bb]], add=True)
            pltpu.sync_copy(puv[bb], accp.at[colv[bb]], add=True)

            @pl.when(cb + 2 < SCPT)
            def _():
                cstart(cb + 2, bb)

        @pl.when(b == 0)
        def _():
            on_buf(0)

        @pl.when(b == 1)
        def _():
            on_buf(1)

        return carry

    lax.fori_loop(0, SCPT, chunk, 0)
    plsc.subcore_barrier()
    pltpu.sync_copy(accx.at[pl.ds(rbase, RPT)], px_hbm.at[cid].at[pl.ds(rbase, RPT)])
    pltpu.sync_copy(accp.at[pl.ds(rbase, RPT)], pp_hbm.at[cid].at[pl.ds(rbase, RPT)])


BE = 1280


def _mlp_body(xr, xc, ea, dr_ref, w1a, w1b, w1c, s1, b1, wx2, bx2, wp2, bp2,
              we2, be2, msg_o, pu_o, eu_o):
    f32 = jnp.float32
    bf16 = jnp.bfloat16
    dr = dr_ref[...]
    h = (jnp.dot(xr[...].astype(bf16), w1a[...], preferred_element_type=f32)
         + jnp.dot(xc[...].astype(bf16), w1b[...], preferred_element_type=f32)
         + jnp.dot(ea[...], w1c[...], preferred_element_type=f32)
         + jnp.dot(dr * dr, s1[...], preferred_element_type=f32)
         + b1[...])
    h = h * jax.nn.sigmoid(h)
    h16 = h.astype(bf16)
    msg_o[...] = jnp.dot(h16[:, :H], wx2[...], preferred_element_type=f32) + bx2[...]
    wp = jnp.dot(h16[:, H:2 * H], wp2[...], preferred_element_type=f32) + bp2[...]
    eu_o[...] = jnp.dot(h16[:, 2 * H:], we2[...], preferred_element_type=f32) + be2[...]
    colid = lax.broadcasted_iota(jnp.int32, (1, G), 1)
    relmask = jnp.where(colid < 3, 1.0, 0.0).astype(f32)
    pu_o[...] = wp * (dr * relmask)


def _full(shape):
    return pl.BlockSpec(shape, lambda i: (0,) * len(shape))


_mlp_call = pl.pallas_call(
    _mlp_body,
    grid=(E // BE,),
    in_specs=[
        pl.BlockSpec((BE, D), lambda i: (i, 0)),
        pl.BlockSpec((BE, D), lambda i: (i, 0)),
        pl.BlockSpec((BE, B), lambda i: (i, 0)),
        pl.BlockSpec((BE, G), lambda i: (i, 0)),
        _full((D, 3 * H)),
        _full((D, 3 * H)),
        _full((B, 3 * H)),
        _full((G, 3 * H)),
        _full((1, 3 * H)),
        _full((H, D)),
        _full((1, D)),
        _full((H, 1)),
        _full((1, 1)),
        _full((H, B)),
        _full((1, B)),
    ],
    out_specs=[
        pl.BlockSpec((BE, D), lambda i: (i, 0)),
        pl.BlockSpec((BE, G), lambda i: (i, 0)),
        pl.BlockSpec((BE, B), lambda i: (i, 0)),
    ],
    out_shape=[
        jax.ShapeDtypeStruct((EP, D), jnp.float32),
        jax.ShapeDtypeStruct((EP, G), jnp.float32),
        jax.ShapeDtypeStruct((E, B), jnp.float32),
    ],
)

BN = 2000


def _combine_body(px, pp, ax_o, ap_o):
    ax_o[...] = px[0] + px[1]
    ap_o[...] = pp[0] + pp[1]


_combine_call = pl.pallas_call(
    _combine_body,
    grid=(N // BN,),
    in_specs=[
        pl.BlockSpec((NC, BN, D), lambda i: (0, i, 0)),
        pl.BlockSpec((NC, BN, G), lambda i: (0, i, 0)),
    ],
    out_specs=[
        pl.BlockSpec((BN, D), lambda i: (i, 0)),
        pl.BlockSpec((BN, G), lambda i: (i, 0)),
    ],
    out_shape=[
        jax.ShapeDtypeStruct((N, D), jnp.float32),
        jax.ShapeDtypeStruct((N, G), jnp.float32),
    ],
)


def kernel(x, pos, edge_index, edge_attr, Wx1, bx1, Wx2, bx2,
           Wp1, bp1, Wp2, bp2, We1, be1, We2, be2):
    pad = EP - E
    rowp = jnp.concatenate([edge_index[0], jnp.zeros((pad,), jnp.int32)])
    colg = jnp.concatenate([edge_index[1], jnp.zeros((pad,), jnp.int32)])
    colp = jnp.concatenate([edge_index[1], jnp.full((pad,), N, jnp.int32)])
    posp = jnp.zeros((N, G), jnp.float32).at[:, :3].set(pos)

    xr, xc, dr = _gather_kernel(x, posp, rowp, colg)

    w1cat = jnp.concatenate([Wx1, Wp1, We1], axis=1)
    b1cat = jnp.concatenate([bx1, bp1, be1]).reshape(1, 3 * H)
    w1a = w1cat[:D]
    w1b = w1cat[D:2 * D]
    w1c = w1cat[2 * D:2 * D + B]
    w1d = w1cat[2 * D + B]
    s1 = jnp.zeros((G, 3 * H), jnp.float32).at[0].set(w1d).at[1].set(w1d).at[2].set(w1d)

    bf16 = jnp.bfloat16
    msg, pu, eu = _mlp_call(
        xr, xc, edge_attr, dr, w1a.astype(bf16), w1b.astype(bf16), w1c, s1,
        b1cat, Wx2.astype(bf16), bx2.reshape(1, D), Wp2.astype(bf16),
        bp2.reshape(1, 1), We2.astype(bf16), be2.reshape(1, B))

    zx = jnp.zeros((NP, D), jnp.float32)
    zp = jnp.zeros((NP, G), jnp.float32)
    px, pp = _scatter_kernel(colp, msg, pu, zx, zp)
    aggx, aggp = _combine_call(px, pp)
    return aggx, aggp[:, :3], eu
